# Initial kernel scaffold; baseline (speedup 1.0000x reference)
import jax
import jax.numpy as jnp
from jax.experimental import pallas as pl


def kernel(x, embedding):
    raise NotImplementedError("write your pallas kernel here")


# speedup vs baseline: 1.8765x; 1.8765x over previous
import jax
import jax.numpy as jnp
from jax import lax
from jax.experimental import pallas as pl
from jax.experimental.pallas import tpu as pltpu
from jax.experimental.pallas import tpu_sc as plsc

_B = 16384 * 50
_D = 64
_CHUNK = 128
_NW = 32
_CPW = _B // (_NW * _CHUNK)
_NBUF = 8
_LAG = 6


def _body(table_hbm, idx_hbm, out_hbm, idx_v, bufs, gsem, ssem):
  c = lax.axis_index("c")
  s = lax.axis_index("s")
  wid = s * 2 + c
  row0 = wid * _CPW
  out0 = row0 * _CHUNK

  pltpu.sync_copy(idx_hbm.at[pl.ds(row0, _CPW)], idx_v)

  def fire_gather(j, b):
    pltpu.async_copy(table_hbm.at[idx_v.at[j]], bufs.at[b], gsem.at[b])

  def wait_gather(j, b):
    pltpu.make_async_copy(table_hbm.at[idx_v.at[j]], bufs.at[b],
                          gsem.at[b]).wait()

  def fire_store(i, b):
    pltpu.async_copy(bufs.at[b], out_hbm.at[pl.ds(out0 + i * _CHUNK, _CHUNK)],
                     ssem.at[b])

  def wait_store(i, b):
    pltpu.make_async_copy(bufs.at[b],
                          out_hbm.at[pl.ds(out0 + i * _CHUNK, _CHUNK)],
                          ssem.at[b]).wait()

  for j in range(_NBUF):
    fire_gather(j, j % _NBUF)
    if j >= _LAG:
      i = j - _LAG
      wait_gather(i, i % _NBUF)
      fire_store(i, i % _NBUF)

  @pl.loop(1, _CPW // _NBUF)
  def _steady(g):
    for b in range(_NBUF):
      j = g * _NBUF + b
      wait_store(j - _NBUF, b)
      fire_gather(j, b)
      i = j - _LAG
      bi = (b - _LAG) % _NBUF
      wait_gather(i, bi)
      fire_store(i, bi)

  for i in range(_CPW - _LAG, _CPW):
    wait_gather(i, i % _NBUF)
    fire_store(i, i % _NBUF)
  for i in range(_CPW - _NBUF, _CPW):
    wait_store(i, i % _NBUF)


@jax.jit
def _gather(table, idx2d):
  mesh = plsc.VectorSubcoreMesh(core_axis_name="c", subcore_axis_name="s")
  run = pl.kernel(
      _body,
      out_type=jax.ShapeDtypeStruct((_B, _D), jnp.float32),
      mesh=mesh,
      compiler_params=pltpu.CompilerParams(use_tc_tiling_on_sc=False),
      scratch_types=[
          pltpu.VMEM((_CPW, _CHUNK), jnp.int32),
          pltpu.VMEM((_NBUF, _CHUNK, _D), jnp.float32),
          pltpu.SemaphoreType.DMA((_NBUF,)),
          pltpu.SemaphoreType.DMA((_NBUF,)),
      ],
  )
  return run(table, idx2d)


def kernel(x, embedding):
  idx2d = x.astype(jnp.int32).reshape(_B // _CHUNK, _CHUNK)
  out = _gather(embedding, idx2d)
  return out.reshape(x.shape[0], x.shape[1], _D)
